# baseline (speedup 1.0000x reference)
import functools

import jax
import jax.numpy as jnp
from jax import lax
from jax.experimental import pallas as pl
from jax.experimental.pallas import tpu as pltpu
from jax.experimental.pallas import tpu_sc as plsc

_LANES = 16
_CH = 16


def _make_emb_kernel(B, V, D):
    info = plsc.get_sparse_core_info()
    nc, ns = info.num_cores, info.num_subcores
    assert D % nc == 0 and B % ns == 0
    dpc = D // nc
    bpt = B // ns
    assert dpc == _LANES and bpt % (2 * _CH) == 0
    nch = bpt // _CH
    mesh = plsc.VectorSubcoreMesh(core_axis_name="c", subcore_axis_name="s")

    @functools.partial(
        pl.kernel,
        mesh=mesh,
        out_type=jax.ShapeDtypeStruct((D, B), jnp.float32),
        scratch_types=[
            pltpu.VMEM((bpt,), jnp.int32),
            pltpu.VMEM((_CH, dpc, 128), jnp.float32),
            pltpu.VMEM((_CH, dpc, 128), jnp.float32),
            pltpu.VMEM((dpc, bpt), jnp.float32),
            pltpu.SemaphoreType.DMA,
            pltpu.SemaphoreType.DMA,
        ],
        compiler_params=pltpu.CompilerParams(
            use_tc_tiling_on_sc=True,
            needs_layout_passes=False,
            disable_bounds_checks=True,
        ),
    )
    def emb(
        idx_hbm, tabt_hbm, outt_hbm, idx_v, buf0, buf1, oblk, sem0, sem1
    ):
        c = lax.axis_index("c")
        s = lax.axis_index("s")
        row0 = pl.multiple_of(c * dpc, dpc)
        row8 = pl.multiple_of(c * dpc + 8, 8)
        base = s * bpt
        pltpu.async_copy(idx_hbm.at[pl.ds(base, bpt)], idx_v, sem0).wait()

        lanes = jax.lax.iota(jnp.int32, _LANES)

        def issue_chunk(k, buf, sem):
            rv = idx_v[pl.ds(k * _CH, _CH)]
            for j in range(_CH):
                start = pl.multiple_of((rv[j] >> 7) << 7, 128)
                pltpu.async_copy(
                    tabt_hbm.at[pl.ds(row0, 8), pl.ds(start, 128)],
                    buf.at[j, pl.ds(0, 8)],
                    sem,
                )
                pltpu.async_copy(
                    tabt_hbm.at[pl.ds(row8, 8), pl.ds(start, 128)],
                    buf.at[j, pl.ds(8, 8)],
                    sem,
                )

        def drain_chunk(k, buf, sem):
            for j in range(_CH):
                pltpu.make_async_copy(
                    tabt_hbm.at[pl.ds(row0, dpc), pl.ds(0, 128)],
                    buf.at[j],
                    sem,
                ).wait()

        def extract_chunk(k, buf):
            rv = idx_v[pl.ds(k * _CH, _CH)]
            for j in range(_CH):
                jr = rv[j] & 127
                col = jnp.full((_LANES,), jr, dtype=jnp.int32)
                v = plsc.load_gather(buf.at[j], [lanes, col])
                out_col = jnp.full((_LANES,), k * _CH + j, dtype=jnp.int32)
                plsc.store_scatter(oblk, [lanes, out_col], v)

        issue_chunk(0, buf0, sem0)
        issue_chunk(1, buf1, sem1)

        def body(m, carry):
            ka = m * 2
            drain_chunk(ka, buf0, sem0)
            extract_chunk(ka, buf0)

            @pl.when(ka + 2 < nch)
            def _():
                issue_chunk(ka + 2, buf0, sem0)

            drain_chunk(ka + 1, buf1, sem1)
            extract_chunk(ka + 1, buf1)

            @pl.when(ka + 3 < nch)
            def _():
                issue_chunk(ka + 3, buf1, sem1)

            return carry

        lax.fori_loop(0, nch // 2, body, 0)
        pltpu.sync_copy(oblk, outt_hbm.at[pl.ds(row0, dpc), pl.ds(base, bpt)])

    return emb


def kernel(y, table):
    (B,) = y.shape
    V, D = table.shape
    emb = _make_emb_kernel(B, V, D)
    out_t = emb(y.astype(jnp.int32), table.T)
    return out_t.T


# speedup vs baseline: 1.0140x; 1.0140x over previous
import functools

import jax
import jax.numpy as jnp
from jax import lax
from jax.experimental import pallas as pl
from jax.experimental.pallas import tpu as pltpu
from jax.experimental.pallas import tpu_sc as plsc

_LANES = 16
_CH = 16


def _make_emb_kernel(B, V, D):
    info = plsc.get_sparse_core_info()
    nc, ns = info.num_cores, info.num_subcores
    assert D % nc == 0 and B % ns == 0
    dpc = D // nc
    bpt = B // ns
    assert dpc == _LANES and bpt % (2 * _CH) == 0
    nch = bpt // _CH
    mesh = plsc.VectorSubcoreMesh(core_axis_name="c", subcore_axis_name="s")

    @functools.partial(
        pl.kernel,
        mesh=mesh,
        out_type=jax.ShapeDtypeStruct((D, B), jnp.float32),
        scratch_types=[
            pltpu.VMEM((bpt,), jnp.int32),
            pltpu.VMEM((_CH, dpc, 128), jnp.float32),
            pltpu.VMEM((_CH, dpc, 128), jnp.float32),
            pltpu.VMEM((dpc, bpt), jnp.float32),
            pltpu.SemaphoreType.DMA,
            pltpu.SemaphoreType.DMA,
        ],
        compiler_params=pltpu.CompilerParams(
            use_tc_tiling_on_sc=True,
            needs_layout_passes=False,
            disable_bounds_checks=True,
        ),
    )
    def emb(
        idx_hbm, tabt_hbm, outt_hbm, idx_v, buf0, buf1, oblk, sem0, sem1
    ):
        c = lax.axis_index("c")
        s = lax.axis_index("s")
        row0 = pl.multiple_of(c * dpc, dpc)
        base = s * bpt
        pltpu.async_copy(idx_hbm.at[pl.ds(base, bpt)], idx_v, sem0).wait()

        lanes = jax.lax.iota(jnp.int32, _LANES)

        def issue_chunk(k, buf, sem):
            rv = idx_v[pl.ds(k * _CH, _CH)]
            for j in range(_CH):
                start = pl.multiple_of((rv[j] >> 7) << 7, 128)
                pltpu.async_copy(
                    tabt_hbm.at[pl.ds(row0, dpc), pl.ds(start, 128)],
                    buf.at[j],
                    sem,
                )

        def drain_chunk(k, buf, sem):
            for j in range(_CH):
                pltpu.make_async_copy(
                    tabt_hbm.at[pl.ds(row0, dpc), pl.ds(0, 128)],
                    buf.at[j],
                    sem,
                ).wait()

        def extract_chunk(k, buf):
            rv = idx_v[pl.ds(k * _CH, _CH)]
            for j in range(_CH):
                jr = rv[j] & 127
                col = jnp.full((_LANES,), jr, dtype=jnp.int32)
                v = plsc.load_gather(buf.at[j], [lanes, col])
                out_col = jnp.full((_LANES,), k * _CH + j, dtype=jnp.int32)
                plsc.store_scatter(oblk, [lanes, out_col], v)

        issue_chunk(0, buf0, sem0)
        issue_chunk(1, buf1, sem1)

        def body(m, carry):
            ka = m * 2
            drain_chunk(ka, buf0, sem0)
            extract_chunk(ka, buf0)

            @pl.when(ka + 2 < nch)
            def _():
                issue_chunk(ka + 2, buf0, sem0)

            drain_chunk(ka + 1, buf1, sem1)
            extract_chunk(ka + 1, buf1)

            @pl.when(ka + 3 < nch)
            def _():
                issue_chunk(ka + 3, buf1, sem1)

            return carry

        lax.fori_loop(0, nch // 2, body, 0)
        pltpu.sync_copy(oblk, outt_hbm.at[pl.ds(row0, dpc), pl.ds(base, bpt)])

    return emb


def kernel(y, table):
    (B,) = y.shape
    V, D = table.shape
    emb = _make_emb_kernel(B, V, D)
    out_t = emb(y.astype(jnp.int32), table.T)
    return out_t.T
